# baseline (device time: 16027 ns/iter reference)
import json
import os

import jax
import jax.numpy as jnp
from jax import lax
from jax.experimental import pallas as pl
from jax.experimental.pallas import tpu as pltpu

N_DEV = 32
BLK = 256


def make_kernel(
    credit_wait=True,
    fix_pass=True,
    recv_wait=True,
    mxu_colsum=True,
    compute_overlap=True,
    manual_in=True,
    collective_id=0,
):
    def kernel(x):
        m, n = x.shape
        nblk = m // BLK

        def body(
            x_ref, out_ref, xf_ref, xb_ref, acc_ref, recv_ref,
            send_sems, recv_sems, in_sem,
        ):
            my = lax.axis_index("i")

            barrier_sem = pltpu.get_barrier_semaphore()
            for i in range(N_DEV):

                @pl.when(i < my)
                def _credit(i=i):
                    pl.semaphore_signal(
                        barrier_sem,
                        inc=1,
                        device_id=(i,),
                        device_id_type=pl.DeviceIdType.MESH,
                    )

            if manual_in:
                copy = pltpu.make_async_copy(x_ref, xf_ref, in_sem)
                copy.start()
                copy.wait()
                src_ref = xf_ref
            else:
                src_ref = x_ref
            xb_ref[:, :] = src_ref[:, :].astype(jnp.bfloat16)
            if mxu_colsum:
                ones8 = jnp.ones((8, m), jnp.bfloat16)
                tot8 = lax.dot(
                    ones8, xb_ref[:, :], preferred_element_type=jnp.float32
                )
                acc_ref[:, :] = tot8[0:1, :]
            else:
                acc_ref[:, :] = jnp.sum(src_ref[:, :], axis=0, keepdims=True)

            if credit_wait:
                for j in range(N_DEV):

                    @pl.when(j > my)
                    def _wait_credit(j=j):
                        pl.semaphore_wait(barrier_sem, 1)

            for j in range(N_DEV):

                @pl.when(j > my)
                def _send(j=j):
                    rdma = pltpu.make_async_remote_copy(
                        src_ref=acc_ref,
                        dst_ref=recv_ref.at[my],
                        send_sem=send_sems.at[j],
                        recv_sem=recv_sems.at[my],
                        device_id=(j,),
                        device_id_type=pl.DeviceIdType.MESH,
                    )
                    rdma.start()

            rows = lax.broadcasted_iota(jnp.int32, (BLK, BLK), 0)
            cols = lax.broadcasted_iota(jnp.int32, (BLK, BLK), 1)
            tri = (rows >= cols).astype(jnp.bfloat16)

            def run_blocks(carry):
                for b in range(nblk):
                    cs = lax.dot(
                        tri,
                        xb_ref[b * BLK:(b + 1) * BLK, :],
                        preferred_element_type=jnp.float32,
                    )
                    out_ref[b * BLK:(b + 1) * BLK, :] = (cs + carry).astype(
                        jnp.bfloat16
                    )
                    carry = carry + cs[BLK - 1:BLK, :]

            if compute_overlap:
                run_blocks(jnp.zeros((1, n), jnp.float32))

            if recv_wait:
                for i in range(N_DEV):

                    @pl.when(i < my)
                    def _recv(i=i):
                        rdma = pltpu.make_async_remote_copy(
                            src_ref=acc_ref,
                            dst_ref=recv_ref.at[i],
                            send_sem=send_sems.at[i],
                            recv_sem=recv_sems.at[i],
                            device_id=(my,),
                            device_id_type=pl.DeviceIdType.MESH,
                        )
                        rdma.wait_recv()

            r = recv_ref[:, 0, :]
            ids = lax.broadcasted_iota(jnp.int32, (N_DEV, n), 0)
            offset = jnp.sum(
                jnp.where(ids < my, r, 0.0), axis=0, keepdims=True
            )

            if compute_overlap:
                if fix_pass:

                    @pl.when(my > 0)
                    def _fix():
                        for b in range(nblk):
                            sl = slice(b * BLK, (b + 1) * BLK)
                            out_ref[sl, :] = (
                                out_ref[sl, :].astype(jnp.float32) + offset
                            ).astype(jnp.bfloat16)

            else:
                run_blocks(offset)

            for j in range(N_DEV):

                @pl.when(j > my)
                def _drain(j=j):
                    rdma = pltpu.make_async_remote_copy(
                        src_ref=acc_ref,
                        dst_ref=recv_ref.at[my],
                        send_sem=send_sems.at[j],
                        recv_sem=recv_sems.at[my],
                        device_id=(j,),
                        device_id_type=pl.DeviceIdType.MESH,
                    )
                    rdma.wait_send()

        return pl.pallas_call(
            body,
            out_shape=jax.ShapeDtypeStruct((m, n), jnp.bfloat16),
            in_specs=[
                pl.BlockSpec(
                    memory_space=pl.ANY if manual_in else pltpu.VMEM
                )
            ],
            out_specs=pl.BlockSpec(memory_space=pltpu.VMEM),
            scratch_shapes=[
                pltpu.VMEM(
                    (m, n) if manual_in else (8, 128), jnp.float32
                ),
                pltpu.VMEM((m, n), jnp.bfloat16),
                pltpu.VMEM((1, n), jnp.float32),
                pltpu.VMEM((N_DEV, 1, n), jnp.float32),
                pltpu.SemaphoreType.DMA((N_DEV,)),
                pltpu.SemaphoreType.DMA((N_DEV,)),
                pltpu.SemaphoreType.DMA,
            ],
            compiler_params=pltpu.CompilerParams(collective_id=collective_id),
        )(x)

    return kernel


_OPTS = json.loads(os.environ.get("KERNEL_OPTS", "{}"))
kernel = make_kernel(**_OPTS)


# device time: 14218 ns/iter; 1.1272x vs baseline; 1.1272x over previous
import json
import os

import jax
import jax.numpy as jnp
from jax import lax
from jax.experimental import pallas as pl
from jax.experimental.pallas import tpu as pltpu

N_DEV = 32
BLK = 256


P_SIZE = 8
N_PLANES = 4


def make_hier_kernel(collective_id=0):

    def kernel(x):
        m, n = x.shape
        nblk = m // BLK

        def body(
            x_ref, out_ref, xf_ref, xb_ref, acc_ref, psum_ref,
            islots, pslots, isend_sems, irecv_sems, psend_sems, precv_sems,
            in_sem,
        ):
            my = lax.axis_index("i")
            my_p = my // P_SIZE
            my_ip = my % P_SIZE

            barrier_sem = pltpu.get_barrier_semaphore()
            for s in range(P_SIZE):

                @pl.when(s != my_ip)
                def _ic(s=s):
                    pl.semaphore_signal(
                        barrier_sem,
                        inc=1,
                        device_id=(my_p * P_SIZE + s,),
                        device_id_type=pl.DeviceIdType.MESH,
                    )

            for p in range(N_PLANES):

                @pl.when(p < my_p)
                def _pc(p=p):
                    pl.semaphore_signal(
                        barrier_sem,
                        inc=1,
                        device_id=(p * P_SIZE + my_ip,),
                        device_id_type=pl.DeviceIdType.MESH,
                    )

            copy = pltpu.make_async_copy(x_ref, xf_ref, in_sem)
            copy.start()
            copy.wait()
            xb_ref[:, :] = xf_ref[:, :].astype(jnp.bfloat16)
            ones8 = jnp.ones((8, m), jnp.bfloat16)
            tot8 = lax.dot(
                ones8, xb_ref[:, :], preferred_element_type=jnp.float32
            )
            acc_ref[:, :] = tot8[0:1, :]

            for _ in range(P_SIZE - 1):
                pl.semaphore_wait(barrier_sem, 1)
            for s in range(P_SIZE):

                @pl.when(s != my_ip)
                def _is(s=s):
                    rdma = pltpu.make_async_remote_copy(
                        src_ref=acc_ref,
                        dst_ref=islots.at[my_ip],
                        send_sem=isend_sems.at[s],
                        recv_sem=irecv_sems.at[my_ip],
                        device_id=(my_p * P_SIZE + s,),
                        device_id_type=pl.DeviceIdType.MESH,
                    )
                    rdma.start()

            rows = lax.broadcasted_iota(jnp.int32, (BLK, BLK), 0)
            cols = lax.broadcasted_iota(jnp.int32, (BLK, BLK), 1)
            tri = (rows >= cols).astype(jnp.bfloat16)

            def do_block(b, carry):
                cs = lax.dot(
                    tri,
                    xb_ref[b * BLK:(b + 1) * BLK, :],
                    preferred_element_type=jnp.float32,
                )
                out_ref[b * BLK:(b + 1) * BLK, :] = (cs + carry).astype(
                    jnp.bfloat16
                )
                return carry + cs[BLK - 1:BLK, :]

            carry = jnp.zeros((1, n), jnp.float32)
            for b in range(nblk // 2):
                carry = do_block(b, carry)

            for s in range(P_SIZE):

                @pl.when(s != my_ip)
                def _ir(s=s):
                    rdma = pltpu.make_async_remote_copy(
                        src_ref=acc_ref,
                        dst_ref=islots.at[s],
                        send_sem=isend_sems.at[s],
                        recv_sem=irecv_sems.at[s],
                        device_id=(my,),
                        device_id_type=pl.DeviceIdType.MESH,
                    )
                    rdma.wait_recv()

            ivals = islots[:, 0, :]
            ip_ids = lax.broadcasted_iota(jnp.int32, (P_SIZE, n), 0)
            intra_off = jnp.sum(
                jnp.where(ip_ids < my_ip, ivals, 0.0), axis=0, keepdims=True
            )
            psum_ref[:, :] = acc_ref[:, :] + jnp.sum(
                jnp.where(ip_ids != my_ip, ivals, 0.0), axis=0, keepdims=True
            )

            for p in range(N_PLANES):

                @pl.when(p > my_p)
                def _pw(p=p):
                    pl.semaphore_wait(barrier_sem, 1)

            for p in range(N_PLANES):

                @pl.when(p > my_p)
                def _ps(p=p):
                    rdma = pltpu.make_async_remote_copy(
                        src_ref=psum_ref,
                        dst_ref=pslots.at[my_p],
                        send_sem=psend_sems.at[p],
                        recv_sem=precv_sems.at[my_p],
                        device_id=(p * P_SIZE + my_ip,),
                        device_id_type=pl.DeviceIdType.MESH,
                    )
                    rdma.start()

            for b in range(nblk // 2, nblk):
                carry = do_block(b, carry)

            for p in range(N_PLANES):

                @pl.when(p < my_p)
                def _pr(p=p):
                    rdma = pltpu.make_async_remote_copy(
                        src_ref=psum_ref,
                        dst_ref=pslots.at[p],
                        send_sem=psend_sems.at[p],
                        recv_sem=precv_sems.at[p],
                        device_id=(my,),
                        device_id_type=pl.DeviceIdType.MESH,
                    )
                    rdma.wait_recv()

            pvals = pslots[:, 0, :]
            p_ids = lax.broadcasted_iota(jnp.int32, (N_PLANES, n), 0)
            offset = intra_off + jnp.sum(
                jnp.where(p_ids < my_p, pvals, 0.0), axis=0, keepdims=True
            )

            @pl.when(my > 0)
            def _fix():
                for b in range(nblk):
                    sl = slice(b * BLK, (b + 1) * BLK)
                    out_ref[sl, :] = (
                        out_ref[sl, :].astype(jnp.float32) + offset
                    ).astype(jnp.bfloat16)

            for s in range(P_SIZE):

                @pl.when(s != my_ip)
                def _id(s=s):
                    rdma = pltpu.make_async_remote_copy(
                        src_ref=acc_ref,
                        dst_ref=islots.at[my_ip],
                        send_sem=isend_sems.at[s],
                        recv_sem=irecv_sems.at[my_ip],
                        device_id=(my,),
                        device_id_type=pl.DeviceIdType.MESH,
                    )
                    rdma.wait_send()

            for p in range(N_PLANES):

                @pl.when(p > my_p)
                def _pd(p=p):
                    rdma = pltpu.make_async_remote_copy(
                        src_ref=psum_ref,
                        dst_ref=pslots.at[my_p],
                        send_sem=psend_sems.at[p],
                        recv_sem=precv_sems.at[my_p],
                        device_id=(my,),
                        device_id_type=pl.DeviceIdType.MESH,
                    )
                    rdma.wait_send()

        return pl.pallas_call(
            body,
            out_shape=jax.ShapeDtypeStruct((m, n), jnp.bfloat16),
            in_specs=[pl.BlockSpec(memory_space=pl.ANY)],
            out_specs=pl.BlockSpec(memory_space=pltpu.VMEM),
            scratch_shapes=[
                pltpu.VMEM((m, n), jnp.float32),
                pltpu.VMEM((m, n), jnp.bfloat16),
                pltpu.VMEM((1, n), jnp.float32),
                pltpu.VMEM((1, n), jnp.float32),
                pltpu.VMEM((P_SIZE, 1, n), jnp.float32),
                pltpu.VMEM((N_PLANES, 1, n), jnp.float32),
                pltpu.SemaphoreType.DMA((P_SIZE,)),
                pltpu.SemaphoreType.DMA((P_SIZE,)),
                pltpu.SemaphoreType.DMA((N_PLANES,)),
                pltpu.SemaphoreType.DMA((N_PLANES,)),
                pltpu.SemaphoreType.DMA,
            ],
            compiler_params=pltpu.CompilerParams(collective_id=collective_id),
        )(x)

    return kernel


def make_kernel(
    credit_wait=True,
    fix_pass=True,
    recv_wait=True,
    mxu_colsum=True,
    compute_overlap=True,
    manual_in=True,
    hier=True,
    collective_id=0,
):
    if hier:
        return make_hier_kernel(collective_id=collective_id)

    def kernel(x):
        m, n = x.shape
        nblk = m // BLK

        def body(
            x_ref, out_ref, xf_ref, xb_ref, acc_ref, recv_ref,
            send_sems, recv_sems, in_sem,
        ):
            my = lax.axis_index("i")

            barrier_sem = pltpu.get_barrier_semaphore()
            for i in range(N_DEV):

                @pl.when(i < my)
                def _credit(i=i):
                    pl.semaphore_signal(
                        barrier_sem,
                        inc=1,
                        device_id=(i,),
                        device_id_type=pl.DeviceIdType.MESH,
                    )

            if manual_in:
                copy = pltpu.make_async_copy(x_ref, xf_ref, in_sem)
                copy.start()
                copy.wait()
                src_ref = xf_ref
            else:
                src_ref = x_ref
            xb_ref[:, :] = src_ref[:, :].astype(jnp.bfloat16)
            if mxu_colsum:
                ones8 = jnp.ones((8, m), jnp.bfloat16)
                tot8 = lax.dot(
                    ones8, xb_ref[:, :], preferred_element_type=jnp.float32
                )
                acc_ref[:, :] = tot8[0:1, :]
            else:
                acc_ref[:, :] = jnp.sum(src_ref[:, :], axis=0, keepdims=True)

            if credit_wait:
                for j in range(N_DEV):

                    @pl.when(j > my)
                    def _wait_credit(j=j):
                        pl.semaphore_wait(barrier_sem, 1)

            for j in range(N_DEV):

                @pl.when(j > my)
                def _send(j=j):
                    rdma = pltpu.make_async_remote_copy(
                        src_ref=acc_ref,
                        dst_ref=recv_ref.at[my],
                        send_sem=send_sems.at[j],
                        recv_sem=recv_sems.at[my],
                        device_id=(j,),
                        device_id_type=pl.DeviceIdType.MESH,
                    )
                    rdma.start()

            rows = lax.broadcasted_iota(jnp.int32, (BLK, BLK), 0)
            cols = lax.broadcasted_iota(jnp.int32, (BLK, BLK), 1)
            tri = (rows >= cols).astype(jnp.bfloat16)

            def run_blocks(carry):
                for b in range(nblk):
                    cs = lax.dot(
                        tri,
                        xb_ref[b * BLK:(b + 1) * BLK, :],
                        preferred_element_type=jnp.float32,
                    )
                    out_ref[b * BLK:(b + 1) * BLK, :] = (cs + carry).astype(
                        jnp.bfloat16
                    )
                    carry = carry + cs[BLK - 1:BLK, :]

            if compute_overlap:
                run_blocks(jnp.zeros((1, n), jnp.float32))

            if recv_wait:
                for i in range(N_DEV):

                    @pl.when(i < my)
                    def _recv(i=i):
                        rdma = pltpu.make_async_remote_copy(
                            src_ref=acc_ref,
                            dst_ref=recv_ref.at[i],
                            send_sem=send_sems.at[i],
                            recv_sem=recv_sems.at[i],
                            device_id=(my,),
                            device_id_type=pl.DeviceIdType.MESH,
                        )
                        rdma.wait_recv()

            r = recv_ref[:, 0, :]
            ids = lax.broadcasted_iota(jnp.int32, (N_DEV, n), 0)
            offset = jnp.sum(
                jnp.where(ids < my, r, 0.0), axis=0, keepdims=True
            )

            if compute_overlap:
                if fix_pass:

                    @pl.when(my > 0)
                    def _fix():
                        for b in range(nblk):
                            sl = slice(b * BLK, (b + 1) * BLK)
                            out_ref[sl, :] = (
                                out_ref[sl, :].astype(jnp.float32) + offset
                            ).astype(jnp.bfloat16)

            else:
                run_blocks(offset)

            for j in range(N_DEV):

                @pl.when(j > my)
                def _drain(j=j):
                    rdma = pltpu.make_async_remote_copy(
                        src_ref=acc_ref,
                        dst_ref=recv_ref.at[my],
                        send_sem=send_sems.at[j],
                        recv_sem=recv_sems.at[my],
                        device_id=(j,),
                        device_id_type=pl.DeviceIdType.MESH,
                    )
                    rdma.wait_send()

        return pl.pallas_call(
            body,
            out_shape=jax.ShapeDtypeStruct((m, n), jnp.bfloat16),
            in_specs=[
                pl.BlockSpec(
                    memory_space=pl.ANY if manual_in else pltpu.VMEM
                )
            ],
            out_specs=pl.BlockSpec(memory_space=pltpu.VMEM),
            scratch_shapes=[
                pltpu.VMEM(
                    (m, n) if manual_in else (8, 128), jnp.float32
                ),
                pltpu.VMEM((m, n), jnp.bfloat16),
                pltpu.VMEM((1, n), jnp.float32),
                pltpu.VMEM((N_DEV, 1, n), jnp.float32),
                pltpu.SemaphoreType.DMA((N_DEV,)),
                pltpu.SemaphoreType.DMA((N_DEV,)),
                pltpu.SemaphoreType.DMA,
            ],
            compiler_params=pltpu.CompilerParams(collective_id=collective_id),
        )(x)

    return kernel


_OPTS = json.loads(os.environ.get("KERNEL_OPTS", "{}"))
kernel = make_kernel(**_OPTS)


# device time: 14199 ns/iter; 1.1287x vs baseline; 1.0013x over previous
import json
import os

import jax
import jax.numpy as jnp
from jax import lax
from jax.experimental import pallas as pl
from jax.experimental.pallas import tpu as pltpu

N_DEV = 32
BLK = 256


P_SIZE = 8
N_PLANES = 4


def make_hier_kernel(collective_id=0):

    def kernel(x):
        m, n = x.shape
        nblk = m // BLK

        def body(
            x_ref, out_ref, xf_ref, xb_ref, acc_ref, psum_ref,
            islots, pslots, isend_sems, irecv_sems, psend_sems, precv_sems,
            in_sem, p2_sem,
        ):
            my = lax.axis_index("i")
            my_p = my // P_SIZE
            my_ip = my % P_SIZE

            barrier_sem = pltpu.get_barrier_semaphore()
            for s in range(P_SIZE):

                @pl.when(s != my_ip)
                def _ic(s=s):
                    pl.semaphore_signal(
                        barrier_sem,
                        inc=1,
                        device_id=(my_p * P_SIZE + s,),
                        device_id_type=pl.DeviceIdType.MESH,
                    )

            for p in range(N_PLANES):

                @pl.when(p < my_p)
                def _pc(p=p):
                    pl.semaphore_signal(
                        p2_sem,
                        inc=1,
                        device_id=(p * P_SIZE + my_ip,),
                        device_id_type=pl.DeviceIdType.MESH,
                    )

            copy = pltpu.make_async_copy(x_ref, xf_ref, in_sem)
            copy.start()
            copy.wait()
            xb_ref[:, :] = xf_ref[:, :].astype(jnp.bfloat16)
            ones8 = jnp.ones((8, m), jnp.bfloat16)
            tot8 = lax.dot(
                ones8, xb_ref[:, :], preferred_element_type=jnp.float32
            )
            acc_ref[:, :] = tot8[0:1, :]

            for _ in range(P_SIZE - 1):
                pl.semaphore_wait(barrier_sem, 1)
            for s in range(P_SIZE):

                @pl.when(s != my_ip)
                def _is(s=s):
                    rdma = pltpu.make_async_remote_copy(
                        src_ref=acc_ref,
                        dst_ref=islots.at[my_ip],
                        send_sem=isend_sems.at[s],
                        recv_sem=irecv_sems.at[my_ip],
                        device_id=(my_p * P_SIZE + s,),
                        device_id_type=pl.DeviceIdType.MESH,
                    )
                    rdma.start()

            rows = lax.broadcasted_iota(jnp.int32, (BLK, BLK), 0)
            cols = lax.broadcasted_iota(jnp.int32, (BLK, BLK), 1)
            tri = (rows >= cols).astype(jnp.bfloat16)

            def do_block(b, carry):
                cs = lax.dot(
                    tri,
                    xb_ref[b * BLK:(b + 1) * BLK, :],
                    preferred_element_type=jnp.float32,
                )
                out_ref[b * BLK:(b + 1) * BLK, :] = (cs + carry).astype(
                    jnp.bfloat16
                )
                return carry + cs[BLK - 1:BLK, :]

            carry = jnp.zeros((1, n), jnp.float32)
            for b in range(nblk // 2):
                carry = do_block(b, carry)

            for s in range(P_SIZE):

                @pl.when(s != my_ip)
                def _ir(s=s):
                    rdma = pltpu.make_async_remote_copy(
                        src_ref=acc_ref,
                        dst_ref=islots.at[s],
                        send_sem=isend_sems.at[s],
                        recv_sem=irecv_sems.at[s],
                        device_id=(my,),
                        device_id_type=pl.DeviceIdType.MESH,
                    )
                    rdma.wait_recv()

            ivals = islots[:, 0, :]
            ip_ids = lax.broadcasted_iota(jnp.int32, (P_SIZE, n), 0)
            intra_off = jnp.sum(
                jnp.where(ip_ids < my_ip, ivals, 0.0), axis=0, keepdims=True
            )
            psum_ref[:, :] = acc_ref[:, :] + jnp.sum(
                jnp.where(ip_ids != my_ip, ivals, 0.0), axis=0, keepdims=True
            )

            for p in range(N_PLANES):

                @pl.when(p > my_p)
                def _pw(p=p):
                    pl.semaphore_wait(p2_sem, 1)

            for p in range(N_PLANES):

                @pl.when(p > my_p)
                def _ps(p=p):
                    rdma = pltpu.make_async_remote_copy(
                        src_ref=psum_ref,
                        dst_ref=pslots.at[my_p],
                        send_sem=psend_sems.at[p],
                        recv_sem=precv_sems.at[my_p],
                        device_id=(p * P_SIZE + my_ip,),
                        device_id_type=pl.DeviceIdType.MESH,
                    )
                    rdma.start()

            for b in range(nblk // 2, nblk):
                carry = do_block(b, carry)

            for p in range(N_PLANES):

                @pl.when(p < my_p)
                def _pr(p=p):
                    rdma = pltpu.make_async_remote_copy(
                        src_ref=psum_ref,
                        dst_ref=pslots.at[p],
                        send_sem=psend_sems.at[p],
                        recv_sem=precv_sems.at[p],
                        device_id=(my,),
                        device_id_type=pl.DeviceIdType.MESH,
                    )
                    rdma.wait_recv()

            pvals = pslots[:, 0, :]
            p_ids = lax.broadcasted_iota(jnp.int32, (N_PLANES, n), 0)
            offset = intra_off + jnp.sum(
                jnp.where(p_ids < my_p, pvals, 0.0), axis=0, keepdims=True
            )

            @pl.when(my > 0)
            def _fix():
                for b in range(nblk):
                    sl = slice(b * BLK, (b + 1) * BLK)
                    out_ref[sl, :] = (
                        out_ref[sl, :].astype(jnp.float32) + offset
                    ).astype(jnp.bfloat16)

            for s in range(P_SIZE):

                @pl.when(s != my_ip)
                def _id(s=s):
                    rdma = pltpu.make_async_remote_copy(
                        src_ref=acc_ref,
                        dst_ref=islots.at[my_ip],
                        send_sem=isend_sems.at[s],
                        recv_sem=irecv_sems.at[my_ip],
                        device_id=(my,),
                        device_id_type=pl.DeviceIdType.MESH,
                    )
                    rdma.wait_send()

            for p in range(N_PLANES):

                @pl.when(p > my_p)
                def _pd(p=p):
                    rdma = pltpu.make_async_remote_copy(
                        src_ref=psum_ref,
                        dst_ref=pslots.at[my_p],
                        send_sem=psend_sems.at[p],
                        recv_sem=precv_sems.at[my_p],
                        device_id=(my,),
                        device_id_type=pl.DeviceIdType.MESH,
                    )
                    rdma.wait_send()

        return pl.pallas_call(
            body,
            out_shape=jax.ShapeDtypeStruct((m, n), jnp.bfloat16),
            in_specs=[pl.BlockSpec(memory_space=pl.ANY)],
            out_specs=pl.BlockSpec(memory_space=pltpu.VMEM),
            scratch_shapes=[
                pltpu.VMEM((m, n), jnp.float32),
                pltpu.VMEM((m, n), jnp.bfloat16),
                pltpu.VMEM((1, n), jnp.float32),
                pltpu.VMEM((1, n), jnp.float32),
                pltpu.VMEM((P_SIZE, 1, n), jnp.float32),
                pltpu.VMEM((N_PLANES, 1, n), jnp.float32),
                pltpu.SemaphoreType.DMA((P_SIZE,)),
                pltpu.SemaphoreType.DMA((P_SIZE,)),
                pltpu.SemaphoreType.DMA((N_PLANES,)),
                pltpu.SemaphoreType.DMA((N_PLANES,)),
                pltpu.SemaphoreType.DMA,
                pltpu.SemaphoreType.REGULAR,
            ],
            compiler_params=pltpu.CompilerParams(collective_id=collective_id),
        )(x)

    return kernel


def make_kernel(
    credit_wait=True,
    fix_pass=True,
    recv_wait=True,
    mxu_colsum=True,
    compute_overlap=True,
    manual_in=True,
    hier=True,
    collective_id=0,
):
    if hier:
        return make_hier_kernel(collective_id=collective_id)

    def kernel(x):
        m, n = x.shape
        nblk = m // BLK

        def body(
            x_ref, out_ref, xf_ref, xb_ref, acc_ref, recv_ref,
            send_sems, recv_sems, in_sem,
        ):
            my = lax.axis_index("i")

            barrier_sem = pltpu.get_barrier_semaphore()
            for i in range(N_DEV):

                @pl.when(i < my)
                def _credit(i=i):
                    pl.semaphore_signal(
                        barrier_sem,
                        inc=1,
                        device_id=(i,),
                        device_id_type=pl.DeviceIdType.MESH,
                    )

            if manual_in:
                copy = pltpu.make_async_copy(x_ref, xf_ref, in_sem)
                copy.start()
                copy.wait()
                src_ref = xf_ref
            else:
                src_ref = x_ref
            xb_ref[:, :] = src_ref[:, :].astype(jnp.bfloat16)
            if mxu_colsum:
                ones8 = jnp.ones((8, m), jnp.bfloat16)
                tot8 = lax.dot(
                    ones8, xb_ref[:, :], preferred_element_type=jnp.float32
                )
                acc_ref[:, :] = tot8[0:1, :]
            else:
                acc_ref[:, :] = jnp.sum(src_ref[:, :], axis=0, keepdims=True)

            if credit_wait:
                for j in range(N_DEV):

                    @pl.when(j > my)
                    def _wait_credit(j=j):
                        pl.semaphore_wait(barrier_sem, 1)

            for j in range(N_DEV):

                @pl.when(j > my)
                def _send(j=j):
                    rdma = pltpu.make_async_remote_copy(
                        src_ref=acc_ref,
                        dst_ref=recv_ref.at[my],
                        send_sem=send_sems.at[j],
                        recv_sem=recv_sems.at[my],
                        device_id=(j,),
                        device_id_type=pl.DeviceIdType.MESH,
                    )
                    rdma.start()

            rows = lax.broadcasted_iota(jnp.int32, (BLK, BLK), 0)
            cols = lax.broadcasted_iota(jnp.int32, (BLK, BLK), 1)
            tri = (rows >= cols).astype(jnp.bfloat16)

            def run_blocks(carry):
                for b in range(nblk):
                    cs = lax.dot(
                        tri,
                        xb_ref[b * BLK:(b + 1) * BLK, :],
                        preferred_element_type=jnp.float32,
                    )
                    out_ref[b * BLK:(b + 1) * BLK, :] = (cs + carry).astype(
                        jnp.bfloat16
                    )
                    carry = carry + cs[BLK - 1:BLK, :]

            if compute_overlap:
                run_blocks(jnp.zeros((1, n), jnp.float32))

            if recv_wait:
                for i in range(N_DEV):

                    @pl.when(i < my)
                    def _recv(i=i):
                        rdma = pltpu.make_async_remote_copy(
                            src_ref=acc_ref,
                            dst_ref=recv_ref.at[i],
                            send_sem=send_sems.at[i],
                            recv_sem=recv_sems.at[i],
                            device_id=(my,),
                            device_id_type=pl.DeviceIdType.MESH,
                        )
                        rdma.wait_recv()

            r = recv_ref[:, 0, :]
            ids = lax.broadcasted_iota(jnp.int32, (N_DEV, n), 0)
            offset = jnp.sum(
                jnp.where(ids < my, r, 0.0), axis=0, keepdims=True
            )

            if compute_overlap:
                if fix_pass:

                    @pl.when(my > 0)
                    def _fix():
                        for b in range(nblk):
                            sl = slice(b * BLK, (b + 1) * BLK)
                            out_ref[sl, :] = (
                                out_ref[sl, :].astype(jnp.float32) + offset
                            ).astype(jnp.bfloat16)

            else:
                run_blocks(offset)

            for j in range(N_DEV):

                @pl.when(j > my)
                def _drain(j=j):
                    rdma = pltpu.make_async_remote_copy(
                        src_ref=acc_ref,
                        dst_ref=recv_ref.at[my],
                        send_sem=send_sems.at[j],
                        recv_sem=recv_sems.at[my],
                        device_id=(j,),
                        device_id_type=pl.DeviceIdType.MESH,
                    )
                    rdma.wait_send()

        return pl.pallas_call(
            body,
            out_shape=jax.ShapeDtypeStruct((m, n), jnp.bfloat16),
            in_specs=[
                pl.BlockSpec(
                    memory_space=pl.ANY if manual_in else pltpu.VMEM
                )
            ],
            out_specs=pl.BlockSpec(memory_space=pltpu.VMEM),
            scratch_shapes=[
                pltpu.VMEM(
                    (m, n) if manual_in else (8, 128), jnp.float32
                ),
                pltpu.VMEM((m, n), jnp.bfloat16),
                pltpu.VMEM((1, n), jnp.float32),
                pltpu.VMEM((N_DEV, 1, n), jnp.float32),
                pltpu.SemaphoreType.DMA((N_DEV,)),
                pltpu.SemaphoreType.DMA((N_DEV,)),
                pltpu.SemaphoreType.DMA,
            ],
            compiler_params=pltpu.CompilerParams(collective_id=collective_id),
        )(x)

    return kernel


_OPTS = json.loads(os.environ.get("KERNEL_OPTS", "{}"))
kernel = make_kernel(**_OPTS)
